# Initial kernel scaffold; baseline (speedup 1.0000x reference)
import jax
import jax.numpy as jnp
from jax.experimental import pallas as pl


def kernel(x, enh_w1, enh_gamma, enh_beta, enh_w2, w, b):
    raise NotImplementedError("write your pallas kernel here")


# speedup vs baseline: 4.4877x; 4.4877x over previous
import functools

import numpy as np
import jax
import jax.numpy as jnp
from jax import lax
from jax.experimental import pallas as pl
from jax.experimental.pallas import tpu as pltpu


_PAR1 = pltpu.CompilerParams(dimension_semantics=("parallel",))


def _enhancer_kernel(x_ref, w1_ref, s_ref, g_ref, bta_ref, w2_ref, o_ref):
    C, T, HW = x_ref.shape[1], x_ref.shape[2], x_ref.shape[3]
    k = s_ref.shape[0]

    x = x_ref[0].astype(jnp.float32)
    pooled = jnp.sum(x, axis=2) * (1.0 / float(HW))

    shifts = [jnp.dot(pooled, s_ref[j], preferred_element_type=jnp.float32)
              for j in range(k)]
    pstack = jnp.concatenate(shifts, axis=0)
    acc = jnp.dot(w1_ref[...], pstack, preferred_element_type=jnp.float32)

    n = float(C * T)
    mu = jnp.sum(acc) * (1.0 / n)
    d = acc - mu
    var = jnp.sum(d * d) * (1.0 / n)
    y = jnp.tanh(d * lax.rsqrt(var + 1e-5) * g_ref[...] + bta_ref[...])

    act = jax.nn.sigmoid(jnp.dot(w2_ref[...], y, preferred_element_type=jnp.float32))
    o_ref[0] = (x * act[:, :, None]).astype(o_ref.dtype)


def _shift_conv_kernel(f0_ref, w_ref, b_ref, o_ref, *, T, H, W, fold, kt):
    THW, C = f0_ref.shape[1], f0_ref.shape[2]
    HW = H * W
    f0 = f0_ref[0].astype(jnp.float32)

    lane = lax.broadcasted_iota(jnp.int32, (THW, C), 1)
    sh = kt * HW
    zpad = jnp.zeros((sh, C), jnp.float32)
    up = jnp.concatenate([f0[sh:], zpad], axis=0)
    dn = jnp.concatenate([zpad, f0[:THW - sh]], axis=0)
    f1 = jnp.where(lane < fold, up, jnp.where(lane < 2 * fold, dn, f0))

    row = lax.broadcasted_iota(jnp.int32, (THW, C), 0)
    hh = (row // W) % H
    ww = row % W
    parts = []
    for dh in (-1, 0, 1):
        for dw in (-1, 0, 1):
            s = dh * W + dw
            if s > 0:
                shf = jnp.concatenate(
                    [f1[s:], jnp.zeros((s, C), jnp.float32)], axis=0)
            elif s < 0:
                shf = jnp.concatenate(
                    [jnp.zeros((-s, C), jnp.float32), f1[:THW + s]], axis=0)
            else:
                shf = f1
            valid = ((hh + dh >= 0) & (hh + dh < H)
                     & (ww + dw >= 0) & (ww + dw < W))
            parts.append(jnp.where(valid, shf, 0.0))
    patches = jnp.concatenate(parts, axis=1)

    acc = jnp.dot(patches, w_ref[...], preferred_element_type=jnp.float32)
    y = jnp.tanh(acc + b_ref[...].astype(jnp.float32)) + f0
    o_ref[0] = y.astype(o_ref.dtype)


def kernel(x, enh_w1, enh_gamma, enh_beta, enh_w2, w, b):
    B, T, C, H, W = x.shape
    HW, THW = H * W, T * H * W
    k = enh_w1.shape[2]
    pad = (k - 1) // 2

    S = np.zeros((k, T, T), np.float32)
    for j in range(k):
        for u in range(T):
            t = u + pad - j
            if 0 <= t < T:
                S[j, u, t] = 1.0
    S = jnp.asarray(S)
    w1cat = jnp.transpose(enh_w1, (0, 2, 1)).reshape(C, k * C)
    w2m = enh_w2[:, :, 0]

    x_ct = jnp.transpose(x, (0, 2, 1, 3, 4)).reshape(B, C, T, HW)
    m = pl.pallas_call(
        _enhancer_kernel,
        out_shape=jax.ShapeDtypeStruct((B, C, T, HW), x.dtype),
        grid=(B,),
        in_specs=[pl.BlockSpec((1, C, T, HW), lambda i: (i, 0, 0, 0)),
                  pl.BlockSpec((C, k * C), lambda i: (0, 0)),
                  pl.BlockSpec((k, T, T), lambda i: (0, 0, 0)),
                  pl.BlockSpec((C, T), lambda i: (0, 0)),
                  pl.BlockSpec((C, T), lambda i: (0, 0)),
                  pl.BlockSpec((C, C), lambda i: (0, 0))],
        out_specs=pl.BlockSpec((1, C, T, HW), lambda i: (i, 0, 0, 0)),
        compiler_params=_PAR1,
    )(x_ct, w1cat, S, enh_gamma, enh_beta, w2m)

    f0 = m.reshape(B, T, C, HW)
    f0_cl = jnp.transpose(f0, (0, 1, 3, 2)).reshape(B, THW, C)

    wim = jnp.transpose(w, (2, 3, 1, 0)).reshape(9 * C, C)
    b2d = b.reshape(1, C)

    fold = C // 3
    kt = int(np.floor(T * 0.25))
    body = functools.partial(_shift_conv_kernel, T=T, H=H, W=W, fold=fold, kt=kt)
    out_cl = pl.pallas_call(
        body,
        out_shape=jax.ShapeDtypeStruct((B, THW, C), x.dtype),
        grid=(B,),
        in_specs=[pl.BlockSpec((1, THW, C), lambda i: (i, 0, 0)),
                  pl.BlockSpec((9 * C, C), lambda i: (0, 0)),
                  pl.BlockSpec((1, C), lambda i: (0, 0))],
        out_specs=pl.BlockSpec((1, THW, C), lambda i: (i, 0, 0)),
        compiler_params=_PAR1,
    )(f0_cl, wim, b2d)

    out = out_cl.reshape(B, T, HW, C)
    return jnp.transpose(out, (0, 1, 3, 2)).reshape(B, T, C, H, W)
